# baseline (speedup 1.0000x reference)
import functools
import jax
import jax.numpy as jnp
import numpy as np
from jax.experimental import pallas as pl
from jax.experimental.pallas import tpu as pltpu

HID = 128; NL = 3; INT_EMB = 64; BASIS_EMB = 8; OUT_EMB = 256
NSPH = 7; NRAD = 6; CUT = 5.0
N_ATOMS = 40000; NB = 10000; NG = 64; NE = 80000

ET = 512
BT = 512
NE_PAD = ((NE + ET - 1) // ET) * ET
NB_PAD = ((NB + BT - 1) // BT) * BT


def _silu(x):
    return x * jax.nn.sigmoid(x)


def _cdiv(a, b):
    return (a + b - 1) // b


def _emb_body(hs_ref, hd_ref, rbf_ref, w1_ref, w2_ref, w3_ref, b_ref, o_ref):
    acc = jnp.dot(hs_ref[...], w1_ref[...], preferred_element_type=jnp.float32)
    acc += jnp.dot(hd_ref[...], w2_ref[...], preferred_element_type=jnp.float32)
    acc += jnp.dot(rbf_ref[...], w3_ref[...], preferred_element_type=jnp.float32)
    o_ref[...] = _silu(acc + b_ref[...])


def _emb_kernel(hs, hd, rbf, w, b):
    w1, w2, w3 = w[:HID], w[HID:2 * HID], w[2 * HID:]
    n = hs.shape[0]
    grid = (_cdiv(n, ET),)
    return pl.pallas_call(
        _emb_body,
        grid=grid,
        in_specs=[
            pl.BlockSpec((ET, HID), lambda i: (i, 0)),
            pl.BlockSpec((ET, HID), lambda i: (i, 0)),
            pl.BlockSpec((ET, NRAD), lambda i: (i, 0)),
            pl.BlockSpec((HID, HID), lambda i: (0, 0)),
            pl.BlockSpec((HID, HID), lambda i: (0, 0)),
            pl.BlockSpec((NRAD, HID), lambda i: (0, 0)),
            pl.BlockSpec((1, HID), lambda i: (0, 0)),
        ],
        out_specs=pl.BlockSpec((ET, HID), lambda i: (i, 0)),
        out_shape=jax.ShapeDtypeStruct((n, HID), jnp.float32),
    )(hs, hd, rbf, w1, w2, w3, b.reshape(1, HID))


def _pre_body(x_ref, rbf_ref, wji_ref, bji_ref, wkj_ref, bkj_ref,
              r1_ref, r2_ref, wdn_ref, xji_ref, xd_ref):
    x = x_ref[...]
    rbf_h = jnp.dot(jnp.dot(rbf_ref[...], r1_ref[...],
                            preferred_element_type=jnp.float32), r2_ref[...],
                    preferred_element_type=jnp.float32)
    xji = _silu(jnp.dot(x, wji_ref[...], preferred_element_type=jnp.float32)
                + bji_ref[...])
    xkj = _silu(jnp.dot(x, wkj_ref[...], preferred_element_type=jnp.float32)
                + bkj_ref[...])
    xkj = xkj * rbf_h
    xd_ref[...] = _silu(jnp.dot(xkj, wdn_ref[...],
                                preferred_element_type=jnp.float32))
    xji_ref[...] = xji


def _pre_kernel(x, rbf, p):
    n = x.shape[0]
    grid = (_cdiv(n, ET),)
    return pl.pallas_call(
        _pre_body,
        grid=grid,
        in_specs=[
            pl.BlockSpec((ET, HID), lambda i: (i, 0)),
            pl.BlockSpec((ET, NRAD), lambda i: (i, 0)),
            pl.BlockSpec((HID, HID), lambda i: (0, 0)),
            pl.BlockSpec((1, HID), lambda i: (0, 0)),
            pl.BlockSpec((HID, HID), lambda i: (0, 0)),
            pl.BlockSpec((1, HID), lambda i: (0, 0)),
            pl.BlockSpec((NRAD, BASIS_EMB), lambda i: (0, 0)),
            pl.BlockSpec((BASIS_EMB, HID), lambda i: (0, 0)),
            pl.BlockSpec((HID, INT_EMB), lambda i: (0, 0)),
        ],
        out_specs=[
            pl.BlockSpec((ET, HID), lambda i: (i, 0)),
            pl.BlockSpec((ET, INT_EMB), lambda i: (i, 0)),
        ],
        out_shape=[
            jax.ShapeDtypeStruct((n, HID), jnp.float32),
            jax.ShapeDtypeStruct((n, INT_EMB), jnp.float32),
        ],
    )(x, rbf, p['ji']['w'], p['ji']['b'].reshape(1, HID),
      p['kj']['w'], p['kj']['b'].reshape(1, HID),
      p['rbf1']['w'], p['rbf2']['w'], p['down']['w'])


def _post_body(xji_ref, agg_ref, x_ref, wup_ref,
               b0w_ref, b0b_ref, b1w_ref, b1b_ref,
               skw_ref, skb_ref,
               a0w_ref, a0b_ref, a1w_ref, a1b_ref,
               a2w_ref, a2b_ref, a3w_ref, a3b_ref, o_ref):
    h = xji_ref[...] + _silu(jnp.dot(agg_ref[...], wup_ref[...],
                                     preferred_element_type=jnp.float32))
    t = _silu(jnp.dot(h, b0w_ref[...], preferred_element_type=jnp.float32) + b0b_ref[...])
    h = h + _silu(jnp.dot(t, b1w_ref[...], preferred_element_type=jnp.float32) + b1b_ref[...])
    h = _silu(jnp.dot(h, skw_ref[...], preferred_element_type=jnp.float32) + skb_ref[...]) + x_ref[...]
    t = _silu(jnp.dot(h, a0w_ref[...], preferred_element_type=jnp.float32) + a0b_ref[...])
    h = h + _silu(jnp.dot(t, a1w_ref[...], preferred_element_type=jnp.float32) + a1b_ref[...])
    t = _silu(jnp.dot(h, a2w_ref[...], preferred_element_type=jnp.float32) + a2b_ref[...])
    h = h + _silu(jnp.dot(t, a3w_ref[...], preferred_element_type=jnp.float32) + a3b_ref[...])
    o_ref[...] = h


def _post_kernel(xji, agg, x, p):
    n = x.shape[0]
    grid = (_cdiv(n, ET),)
    hh = lambda: pl.BlockSpec((HID, HID), lambda i: (0, 0))
    bb = lambda: pl.BlockSpec((1, HID), lambda i: (0, 0))
    ws = [p['up']['w'],
          p['before'][0][0]['w'], p['before'][0][0]['b'].reshape(1, HID),
          p['before'][0][1]['w'], p['before'][0][1]['b'].reshape(1, HID),
          p['skip']['w'], p['skip']['b'].reshape(1, HID),
          p['after'][0][0]['w'], p['after'][0][0]['b'].reshape(1, HID),
          p['after'][0][1]['w'], p['after'][0][1]['b'].reshape(1, HID),
          p['after'][1][0]['w'], p['after'][1][0]['b'].reshape(1, HID),
          p['after'][1][1]['w'], p['after'][1][1]['b'].reshape(1, HID)]
    wspecs = [pl.BlockSpec((INT_EMB, HID), lambda i: (0, 0))]
    for k in range(1, len(ws)):
        wspecs.append(hh() if ws[k].shape[0] == HID else bb())
    return pl.pallas_call(
        _post_body,
        grid=grid,
        in_specs=[
            pl.BlockSpec((ET, HID), lambda i: (i, 0)),
            pl.BlockSpec((ET, INT_EMB), lambda i: (i, 0)),
            pl.BlockSpec((ET, HID), lambda i: (i, 0)),
        ] + wspecs,
        out_specs=pl.BlockSpec((ET, HID), lambda i: (i, 0)),
        out_shape=jax.ShapeDtypeStruct((n, HID), jnp.float32),
    )(xji, agg, x, *ws)


def _oute_body(x_ref, rbf_ref, wr_ref, keep_ref, o_ref):
    g = jnp.dot(rbf_ref[...], wr_ref[...], preferred_element_type=jnp.float32)
    o_ref[...] = jnp.where(keep_ref[...] > 0, g * x_ref[...], 0.0)


def _oute_kernel(x, rbf, wr, keepf):
    n = x.shape[0]
    grid = (_cdiv(n, ET),)
    return pl.pallas_call(
        _oute_body,
        grid=grid,
        in_specs=[
            pl.BlockSpec((ET, HID), lambda i: (i, 0)),
            pl.BlockSpec((ET, NRAD), lambda i: (i, 0)),
            pl.BlockSpec((NRAD, HID), lambda i: (0, 0)),
            pl.BlockSpec((ET, 1), lambda i: (i, 0)),
        ],
        out_specs=pl.BlockSpec((ET, HID), lambda i: (i, 0)),
        out_shape=jax.ShapeDtypeStruct((n, HID), jnp.float32),
    )(x, rbf, wr, keepf)


def _outb_body(t_ref, wu_ref, l0w_ref, l0b_ref, l1w_ref, l1b_ref,
               l2w_ref, l2b_ref, wo_ref, o_ref):
    t = jnp.dot(t_ref[...], wu_ref[...], preferred_element_type=jnp.float32)
    t = _silu(jnp.dot(t, l0w_ref[...], preferred_element_type=jnp.float32) + l0b_ref[...])
    t = _silu(jnp.dot(t, l1w_ref[...], preferred_element_type=jnp.float32) + l1b_ref[...])
    t = _silu(jnp.dot(t, l2w_ref[...], preferred_element_type=jnp.float32) + l2b_ref[...])
    o_ref[...] = jnp.dot(t, wo_ref[...], preferred_element_type=jnp.float32)


def _outb_kernel(t, p):
    n = t.shape[0]
    grid = (_cdiv(n, BT),)
    oo = lambda: pl.BlockSpec((OUT_EMB, OUT_EMB), lambda i: (0, 0))
    ob = lambda: pl.BlockSpec((1, OUT_EMB), lambda i: (0, 0))
    return pl.pallas_call(
        _outb_body,
        grid=grid,
        in_specs=[
            pl.BlockSpec((BT, HID), lambda i: (i, 0)),
            pl.BlockSpec((HID, OUT_EMB), lambda i: (0, 0)),
            oo(), ob(), oo(), ob(), oo(), ob(),
            pl.BlockSpec((OUT_EMB, HID), lambda i: (0, 0)),
        ],
        out_specs=pl.BlockSpec((BT, HID), lambda i: (i, 0)),
        out_shape=jax.ShapeDtypeStruct((n, HID), jnp.float32),
    )(t, p['up']['w'],
      p['lins'][0]['w'], p['lins'][0]['b'].reshape(1, OUT_EMB),
      p['lins'][1]['w'], p['lins'][1]['b'].reshape(1, OUT_EMB),
      p['lins'][2]['w'], p['lins'][2]['b'].reshape(1, OUT_EMB),
      p['out']['w'])


RC = 8
NTILES = NE_PAD // ET
TBLW = 80


def _trip_body(xdg_ref, u2g_ref, rbg_ref, srcg_ref, u1_ref, dstf_ref,
               keepf_ref, csf_ref, w12r_ref, r0_ref, acc_ref):
    xdg = xdg_ref[...]
    u2g = u2g_ref[...]
    rbg = rbg_ref[...]
    srcg = srcg_ref[...]
    u1 = u1_ref[...]
    dstf = dstf_ref[...]
    keepf = keepf_ref[...]
    cs = csf_ref[...]
    r_f = r0_ref[0, 0] + jnp.float32(pl.program_id(1))
    c = jnp.sum(u1 * u2g, axis=-1, keepdims=True)
    rbw = jnp.dot(rbg, w12r_ref[...], preferred_element_type=jnp.float32)
    tl0 = jnp.ones_like(c)
    tl1 = c
    sbf_h = rbw[:, 0:INT_EMB] + c * rbw[:, INT_EMB:2 * INT_EMB]
    for l in range(2, NSPH):
        tl0, tl1 = tl1, 2.0 * c * tl1 - tl0
        sbf_h += tl1 * rbw[:, l * INT_EMB:(l + 1) * INT_EMB]
    valid = (keepf > 0) & (r_f < cs) & (srcg != dstf)
    contrib = jnp.where(valid, xdg * sbf_h, 0.0)

    @pl.when(pl.program_id(1) == 0)
    def _():
        acc_ref[...] = contrib

    @pl.when(pl.program_id(1) != 0)
    def _():
        acc_ref[...] = acc_ref[...] + contrib


def _trip_chunk(xdg, u2g, rbg, srcg, u1, dstf, keepf, csf, w12r, r0f):
    gat = lambda w: pl.BlockSpec((ET, w), lambda t, r: (r * NTILES + t, 0))
    sta = lambda w: pl.BlockSpec((ET, w), lambda t, r: (t, 0))
    return pl.pallas_call(
        _trip_body,
        grid=(NTILES, RC),
        in_specs=[
            gat(INT_EMB), gat(4), gat(NRAD), gat(1),
            sta(4), sta(1), sta(1), sta(1),
            pl.BlockSpec((NRAD, NSPH * INT_EMB), lambda t, r: (0, 0)),
            pl.BlockSpec((1, 1), lambda t, r: (0, 0)),
        ],
        out_specs=pl.BlockSpec((ET, INT_EMB), lambda t, r: (t, 0)),
        out_shape=jax.ShapeDtypeStruct((NE_PAD, INT_EMB), jnp.float32),
    )(xdg, u2g, rbg, srcg, u1, dstf, keepf, csf, w12r, r0f)


def _l2n_body(x_ref, o_ref):
    x = x_ref[...]
    nrm = jnp.sqrt(jnp.sum(x * x, axis=-1, keepdims=True))
    o_ref[...] = x / jnp.maximum(nrm, 1e-12)


def _l2n_kernel(x):
    n, d = x.shape
    bt = min(n, BT)
    grid = (_cdiv(n, bt),)
    return pl.pallas_call(
        _l2n_body,
        grid=grid,
        in_specs=[pl.BlockSpec((bt, d), lambda i: (i, 0))],
        out_specs=pl.BlockSpec((bt, d), lambda i: (i, 0)),
        out_shape=jax.ShapeDtypeStruct((n, d), jnp.float32),
    )(x)


def _envelope(x):
    p = 6
    a = -(p + 1) * (p + 2) / 2.0
    b = p * (p + 2)
    cc = -p * (p + 1) / 2.0
    xs = jnp.maximum(x, 1e-9)
    env = 1.0 / xs + a * xs ** (p - 1) + b * xs ** p + cc * xs ** (p + 1)
    return jnp.where(x < 1.0, env, 0.0)


def _rbf(d):
    x = d / CUT
    freqs = jnp.arange(1, NRAD + 1, dtype=jnp.float32) * jnp.pi
    return _envelope(x)[:, None] * jnp.sin(freqs[None, :] * x[:, None])


def _segsum(x, idx, n):
    return jax.ops.segment_sum(x, idx, num_segments=n)


def _pad_rows(x, n):
    return jnp.pad(x, ((0, n - x.shape[0]),) + ((0, 0),) * (x.ndim - 1))


def kernel(H, Z, block_id, batch_id, edges, params):
    H2 = _segsum(H, block_id, NB)
    cnt = _segsum(jnp.ones((N_ATOMS, 1), jnp.float32), block_id, NB)
    cnt = jnp.maximum(cnt, 1.0)
    H2 = H2 / cnt
    Zb = _segsum(Z, block_id, NB) / cnt

    src, dst = edges[0], edges[1]
    dvec = Zb[dst] - Zb[src]
    dist = jnp.sqrt(jnp.sum(dvec * dvec, axis=-1) + 1e-12)
    keep = dist > 0.01
    rbf = _rbf(dist)

    dst_key = jnp.where(keep, dst, NB)
    order = jnp.argsort(dst_key, stable=True)
    counts = _segsum(keep.astype(jnp.int32), dst, NB)
    offsets = jnp.concatenate([jnp.zeros((1,), counts.dtype),
                               jnp.cumsum(counts)[:-1]])
    max_rank = jnp.max(jnp.where(keep, counts[src], 0))
    counts_src = counts[src]
    base = offsets[src]

    u1 = dvec / dist[:, None]
    u2_tab = -u1
    rbf_p = _pad_rows(rbf, NE_PAD)
    srcf_tab = src.astype(jnp.float32)[:, None]
    u2_tab4 = jnp.pad(u2_tab, ((0, 0), (0, 1)))
    keepf = _pad_rows(keep.astype(jnp.float32)[:, None], NE_PAD)
    u1p = _pad_rows(jnp.pad(u1, ((0, 0), (0, 1))), NE_PAD)
    dstf = _pad_rows(dst.astype(jnp.float32)[:, None], NE_PAD)
    csf = _pad_rows(counts_src.astype(jnp.float32)[:, None], NE_PAD)
    base_p = _pad_rows(base[:, None], NE_PAD)[:, 0]

    Hs = _pad_rows(H2[src], NE_PAD)
    Hd = _pad_rows(H2[dst], NE_PAD)
    x = _emb_kernel(Hs, Hd, rbf_p, params['emb']['w'], params['emb']['b'])

    def out_block(p, x):
        y = _oute_kernel(x, rbf_p, p['rbf']['w'], keepf)[:NE]
        t = _segsum(y, dst, NB)
        t = _outb_kernel(_pad_rows(t, NB_PAD), p)[:NB]
        return t

    P = out_block(params['outs'][0], x)

    for b in range(NL):
        p = params['inter'][b]
        xji, xd = _pre_kernel(x, rbf_p, p)
        w12 = jnp.dot(p['sbf1']['w'], p['sbf2']['w'])
        w12r = jnp.transpose(w12.reshape(NSPH, NRAD, INT_EMB),
                             (1, 0, 2)).reshape(NRAD, NSPH * INT_EMB)
        xd_ne = xd[:NE]

        def body(state):
            r0, acc = state
            idx = jnp.clip(base_p[None, :] + r0
                           + jnp.arange(RC, dtype=r0.dtype)[:, None],
                           0, NE - 1)
            kj = order[idx].reshape(-1)
            r0f = r0.astype(jnp.float32).reshape(1, 1)
            acc = acc + _trip_chunk(xd_ne[kj], u2_tab4[kj], rbf[kj],
                                    srcf_tab[kj], u1p, dstf, keepf, csf,
                                    w12r, r0f)
            return r0 + RC, acc

        r0 = jnp.zeros((), max_rank.dtype)
        acc0 = jnp.zeros((NE_PAD, INT_EMB), jnp.float32)
        _, acc = jax.lax.while_loop(lambda s: s[0] < max_rank, body, (r0, acc0))
        x = _post_kernel(xji, acc, x, p)
        P = P + out_block(params['outs'][b + 1], x)

    block_repr = _l2n_kernel(_pad_rows(P, NB_PAD))[:NB]
    graph_repr = _l2n_kernel(_segsum(block_repr, batch_id, NG))
    return (H2, block_repr, graph_repr)


# speedup vs baseline: 2.2915x; 2.2915x over previous
import functools
import jax
import jax.numpy as jnp
import numpy as np
from jax.experimental import pallas as pl
from jax.experimental.pallas import tpu as pltpu
from jax.experimental.pallas import tpu_sc as plsc

HID = 128; NL = 3; INT_EMB = 64; BASIS_EMB = 8; OUT_EMB = 256
NSPH = 7; NRAD = 6; CUT = 5.0
N_ATOMS = 40000; NB = 10000; NG = 64; NE = 80000

ET = 512
BT = 512
NE_PAD = ((NE + ET - 1) // ET) * ET
NB_PAD = ((NB + BT - 1) // BT) * BT


def _silu(x):
    return x * jax.nn.sigmoid(x)


def _cdiv(a, b):
    return (a + b - 1) // b


def _emb_body(hs_ref, hd_ref, rbf_ref, w1_ref, w2_ref, w3_ref, b_ref, o_ref):
    acc = jnp.dot(hs_ref[...], w1_ref[...], preferred_element_type=jnp.float32)
    acc += jnp.dot(hd_ref[...], w2_ref[...], preferred_element_type=jnp.float32)
    acc += jnp.dot(rbf_ref[...], w3_ref[...], preferred_element_type=jnp.float32)
    o_ref[...] = _silu(acc + b_ref[...])


def _emb_kernel(hs, hd, rbf, w, b):
    w1, w2, w3 = w[:HID], w[HID:2 * HID], w[2 * HID:]
    n = hs.shape[0]
    grid = (_cdiv(n, ET),)
    return pl.pallas_call(
        _emb_body,
        grid=grid,
        in_specs=[
            pl.BlockSpec((ET, HID), lambda i: (i, 0)),
            pl.BlockSpec((ET, HID), lambda i: (i, 0)),
            pl.BlockSpec((ET, NRAD), lambda i: (i, 0)),
            pl.BlockSpec((HID, HID), lambda i: (0, 0)),
            pl.BlockSpec((HID, HID), lambda i: (0, 0)),
            pl.BlockSpec((NRAD, HID), lambda i: (0, 0)),
            pl.BlockSpec((1, HID), lambda i: (0, 0)),
        ],
        out_specs=pl.BlockSpec((ET, HID), lambda i: (i, 0)),
        out_shape=jax.ShapeDtypeStruct((n, HID), jnp.float32),
    )(hs, hd, rbf, w1, w2, w3, b.reshape(1, HID))


def _pre_body(x_ref, rbf_ref, wji_ref, bji_ref, wkj_ref, bkj_ref,
              r1_ref, r2_ref, wdn_ref, xji_ref, xd_ref):
    x = x_ref[...]
    rbf_h = jnp.dot(jnp.dot(rbf_ref[...], r1_ref[...],
                            preferred_element_type=jnp.float32), r2_ref[...],
                    preferred_element_type=jnp.float32)
    xji = _silu(jnp.dot(x, wji_ref[...], preferred_element_type=jnp.float32)
                + bji_ref[...])
    xkj = _silu(jnp.dot(x, wkj_ref[...], preferred_element_type=jnp.float32)
                + bkj_ref[...])
    xkj = xkj * rbf_h
    xd_ref[...] = _silu(jnp.dot(xkj, wdn_ref[...],
                                preferred_element_type=jnp.float32))
    xji_ref[...] = xji


def _pre_kernel(x, rbf, p):
    n = x.shape[0]
    grid = (_cdiv(n, ET),)
    return pl.pallas_call(
        _pre_body,
        grid=grid,
        in_specs=[
            pl.BlockSpec((ET, HID), lambda i: (i, 0)),
            pl.BlockSpec((ET, NRAD), lambda i: (i, 0)),
            pl.BlockSpec((HID, HID), lambda i: (0, 0)),
            pl.BlockSpec((1, HID), lambda i: (0, 0)),
            pl.BlockSpec((HID, HID), lambda i: (0, 0)),
            pl.BlockSpec((1, HID), lambda i: (0, 0)),
            pl.BlockSpec((NRAD, BASIS_EMB), lambda i: (0, 0)),
            pl.BlockSpec((BASIS_EMB, HID), lambda i: (0, 0)),
            pl.BlockSpec((HID, INT_EMB), lambda i: (0, 0)),
        ],
        out_specs=[
            pl.BlockSpec((ET, HID), lambda i: (i, 0)),
            pl.BlockSpec((ET, INT_EMB), lambda i: (i, 0)),
        ],
        out_shape=[
            jax.ShapeDtypeStruct((n, HID), jnp.float32),
            jax.ShapeDtypeStruct((n, INT_EMB), jnp.float32),
        ],
    )(x, rbf, p['ji']['w'], p['ji']['b'].reshape(1, HID),
      p['kj']['w'], p['kj']['b'].reshape(1, HID),
      p['rbf1']['w'], p['rbf2']['w'], p['down']['w'])


def _post_body(xji_ref, agg_ref, x_ref, wup_ref,
               b0w_ref, b0b_ref, b1w_ref, b1b_ref,
               skw_ref, skb_ref,
               a0w_ref, a0b_ref, a1w_ref, a1b_ref,
               a2w_ref, a2b_ref, a3w_ref, a3b_ref, o_ref):
    h = xji_ref[...] + _silu(jnp.dot(agg_ref[...], wup_ref[...],
                                     preferred_element_type=jnp.float32))
    t = _silu(jnp.dot(h, b0w_ref[...], preferred_element_type=jnp.float32) + b0b_ref[...])
    h = h + _silu(jnp.dot(t, b1w_ref[...], preferred_element_type=jnp.float32) + b1b_ref[...])
    h = _silu(jnp.dot(h, skw_ref[...], preferred_element_type=jnp.float32) + skb_ref[...]) + x_ref[...]
    t = _silu(jnp.dot(h, a0w_ref[...], preferred_element_type=jnp.float32) + a0b_ref[...])
    h = h + _silu(jnp.dot(t, a1w_ref[...], preferred_element_type=jnp.float32) + a1b_ref[...])
    t = _silu(jnp.dot(h, a2w_ref[...], preferred_element_type=jnp.float32) + a2b_ref[...])
    h = h + _silu(jnp.dot(t, a3w_ref[...], preferred_element_type=jnp.float32) + a3b_ref[...])
    o_ref[...] = h


def _post_kernel(xji, agg, x, p):
    n = x.shape[0]
    grid = (_cdiv(n, ET),)
    hh = lambda: pl.BlockSpec((HID, HID), lambda i: (0, 0))
    bb = lambda: pl.BlockSpec((1, HID), lambda i: (0, 0))
    ws = [p['up']['w'],
          p['before'][0][0]['w'], p['before'][0][0]['b'].reshape(1, HID),
          p['before'][0][1]['w'], p['before'][0][1]['b'].reshape(1, HID),
          p['skip']['w'], p['skip']['b'].reshape(1, HID),
          p['after'][0][0]['w'], p['after'][0][0]['b'].reshape(1, HID),
          p['after'][0][1]['w'], p['after'][0][1]['b'].reshape(1, HID),
          p['after'][1][0]['w'], p['after'][1][0]['b'].reshape(1, HID),
          p['after'][1][1]['w'], p['after'][1][1]['b'].reshape(1, HID)]
    wspecs = [pl.BlockSpec((INT_EMB, HID), lambda i: (0, 0))]
    for k in range(1, len(ws)):
        wspecs.append(hh() if ws[k].shape[0] == HID else bb())
    return pl.pallas_call(
        _post_body,
        grid=grid,
        in_specs=[
            pl.BlockSpec((ET, HID), lambda i: (i, 0)),
            pl.BlockSpec((ET, INT_EMB), lambda i: (i, 0)),
            pl.BlockSpec((ET, HID), lambda i: (i, 0)),
        ] + wspecs,
        out_specs=pl.BlockSpec((ET, HID), lambda i: (i, 0)),
        out_shape=jax.ShapeDtypeStruct((n, HID), jnp.float32),
    )(xji, agg, x, *ws)


def _oute_body(x_ref, rbf_ref, wr_ref, keep_ref, o_ref):
    g = jnp.dot(rbf_ref[...], wr_ref[...], preferred_element_type=jnp.float32)
    o_ref[...] = jnp.where(keep_ref[...] > 0, g * x_ref[...], 0.0)


def _oute_kernel(x, rbf, wr, keepf):
    n = x.shape[0]
    grid = (_cdiv(n, ET),)
    return pl.pallas_call(
        _oute_body,
        grid=grid,
        in_specs=[
            pl.BlockSpec((ET, HID), lambda i: (i, 0)),
            pl.BlockSpec((ET, NRAD), lambda i: (i, 0)),
            pl.BlockSpec((NRAD, HID), lambda i: (0, 0)),
            pl.BlockSpec((ET, 1), lambda i: (i, 0)),
        ],
        out_specs=pl.BlockSpec((ET, HID), lambda i: (i, 0)),
        out_shape=jax.ShapeDtypeStruct((n, HID), jnp.float32),
    )(x, rbf, wr, keepf)


def _outb_body(t_ref, wu_ref, l0w_ref, l0b_ref, l1w_ref, l1b_ref,
               l2w_ref, l2b_ref, wo_ref, o_ref):
    t = jnp.dot(t_ref[...], wu_ref[...], preferred_element_type=jnp.float32)
    t = _silu(jnp.dot(t, l0w_ref[...], preferred_element_type=jnp.float32) + l0b_ref[...])
    t = _silu(jnp.dot(t, l1w_ref[...], preferred_element_type=jnp.float32) + l1b_ref[...])
    t = _silu(jnp.dot(t, l2w_ref[...], preferred_element_type=jnp.float32) + l2b_ref[...])
    o_ref[...] = jnp.dot(t, wo_ref[...], preferred_element_type=jnp.float32)


def _outb_kernel(t, p):
    n = t.shape[0]
    grid = (_cdiv(n, BT),)
    oo = lambda: pl.BlockSpec((OUT_EMB, OUT_EMB), lambda i: (0, 0))
    ob = lambda: pl.BlockSpec((1, OUT_EMB), lambda i: (0, 0))
    return pl.pallas_call(
        _outb_body,
        grid=grid,
        in_specs=[
            pl.BlockSpec((BT, HID), lambda i: (i, 0)),
            pl.BlockSpec((HID, OUT_EMB), lambda i: (0, 0)),
            oo(), ob(), oo(), ob(), oo(), ob(),
            pl.BlockSpec((OUT_EMB, HID), lambda i: (0, 0)),
        ],
        out_specs=pl.BlockSpec((BT, HID), lambda i: (i, 0)),
        out_shape=jax.ShapeDtypeStruct((n, HID), jnp.float32),
    )(t, p['up']['w'],
      p['lins'][0]['w'], p['lins'][0]['b'].reshape(1, OUT_EMB),
      p['lins'][1]['w'], p['lins'][1]['b'].reshape(1, OUT_EMB),
      p['lins'][2]['w'], p['lins'][2]['b'].reshape(1, OUT_EMB),
      p['out']['w'])


RC = 8
NTILES = NE_PAD // ET
_GRAN = 32 * 1024
B_GATH = ((RC * NE_PAD + _GRAN - 1) // _GRAN) * _GRAN
TBLW = 128


def _trip_body(gthr_ref, u1_ref, dstf_ref,
               keepf_ref, csf_ref, w12r_ref, r0_ref, acc_ref):
    g = gthr_ref[...]
    xdg = g[:, :INT_EMB]
    u2g = g[:, INT_EMB:INT_EMB + 3]
    rbg = g[:, INT_EMB + 3:INT_EMB + 3 + NRAD]
    srcg = g[:, INT_EMB + 3 + NRAD:INT_EMB + 4 + NRAD]
    u1 = u1_ref[...]
    dstf = dstf_ref[...]
    keepf = keepf_ref[...]
    cs = csf_ref[...]
    r_f = r0_ref[0, 0] + jnp.float32(pl.program_id(1))
    c = jnp.sum(u1[:, 0:3] * u2g, axis=-1, keepdims=True)
    rbw = jnp.dot(rbg, w12r_ref[...], preferred_element_type=jnp.float32)
    tl0 = jnp.ones_like(c)
    tl1 = c
    sbf_h = rbw[:, 0:INT_EMB] + c * rbw[:, INT_EMB:2 * INT_EMB]
    for l in range(2, NSPH):
        tl0, tl1 = tl1, 2.0 * c * tl1 - tl0
        sbf_h += tl1 * rbw[:, l * INT_EMB:(l + 1) * INT_EMB]
    valid = (keepf > 0) & (r_f < cs) & (srcg != dstf)
    contrib = jnp.where(valid, xdg * sbf_h, 0.0)

    @pl.when(pl.program_id(1) == 0)
    def _():
        acc_ref[...] = contrib

    @pl.when(pl.program_id(1) != 0)
    def _():
        acc_ref[...] = acc_ref[...] + contrib


def _trip_chunk(gthr, u1, dstf, keepf, csf, w12r, r0f):
    sta = lambda w: pl.BlockSpec((ET, w), lambda t, r: (t, 0))
    return pl.pallas_call(
        _trip_body,
        grid=(NTILES, RC),
        in_specs=[
            pl.BlockSpec((ET, TBLW), lambda t, r: (r * NTILES + t, 0)),
            sta(4), sta(1), sta(1), sta(1),
            pl.BlockSpec((NRAD, NSPH * INT_EMB), lambda t, r: (0, 0)),
            pl.BlockSpec((1, 1), lambda t, r: (0, 0)),
        ],
        out_specs=pl.BlockSpec((ET, INT_EMB), lambda t, r: (t, 0)),
        out_shape=jax.ShapeDtypeStruct((NE_PAD, INT_EMB), jnp.float32),
    )(gthr, u1, dstf, keepf, csf, w12r, r0f)


SC_NC, SC_NS = 2, 16
SC_NW = SC_NC * SC_NS
GCH = 128
GK = 4
GOUT = GCH * GK


def _sc_gather(tbl, idx):
    B = idx.shape[0]
    W = tbl.shape[1]
    b_per_w = B // SC_NW
    nouter = b_per_w // GOUT
    mesh = plsc.VectorSubcoreMesh(core_axis_name="c", subcore_axis_name="s")

    @functools.partial(
        pl.kernel, mesh=mesh,
        out_type=jax.ShapeDtypeStruct((B, W), jnp.float32),
        scratch_types=[
            pltpu.VMEM((GOUT,), jnp.int32),
            pltpu.VMEM((GOUT, W), jnp.float32),
            pltpu.SemaphoreType.DMA,
        ],
    )
    def k(tbl_hbm, idx_hbm, out_hbm, idx_v, rows_v, sem):
        wid = jax.lax.axis_index("s") * SC_NC + jax.lax.axis_index("c")
        base = wid * b_per_w

        def body(c, carry):
            off = base + c * GOUT
            pltpu.sync_copy(idx_hbm.at[pl.ds(off, GOUT)], idx_v)
            cps = [pltpu.async_copy(tbl_hbm.at[idx_v.at[pl.ds(b * GCH, GCH)]],
                                    rows_v.at[pl.ds(b * GCH, GCH)], sem)
                   for b in range(GK)]
            for cp in cps:
                cp.wait()
            pltpu.sync_copy(rows_v, out_hbm.at[pl.ds(off, GOUT)])
            return carry

        jax.lax.fori_loop(0, nouter, body, 0)

    return k(tbl, idx)


def _l2n_body(x_ref, o_ref):
    x = x_ref[...]
    nrm = jnp.sqrt(jnp.sum(x * x, axis=-1, keepdims=True))
    o_ref[...] = x / jnp.maximum(nrm, 1e-12)


def _l2n_kernel(x):
    n, d = x.shape
    bt = min(n, BT)
    grid = (_cdiv(n, bt),)
    return pl.pallas_call(
        _l2n_body,
        grid=grid,
        in_specs=[pl.BlockSpec((bt, d), lambda i: (i, 0))],
        out_specs=pl.BlockSpec((bt, d), lambda i: (i, 0)),
        out_shape=jax.ShapeDtypeStruct((n, d), jnp.float32),
    )(x)


def _envelope(x):
    p = 6
    a = -(p + 1) * (p + 2) / 2.0
    b = p * (p + 2)
    cc = -p * (p + 1) / 2.0
    xs = jnp.maximum(x, 1e-9)
    env = 1.0 / xs + a * xs ** (p - 1) + b * xs ** p + cc * xs ** (p + 1)
    return jnp.where(x < 1.0, env, 0.0)


def _rbf(d):
    x = d / CUT
    freqs = jnp.arange(1, NRAD + 1, dtype=jnp.float32) * jnp.pi
    return _envelope(x)[:, None] * jnp.sin(freqs[None, :] * x[:, None])


def _segsum(x, idx, n):
    return jax.ops.segment_sum(x, idx, num_segments=n)


def _pad_rows(x, n):
    return jnp.pad(x, ((0, n - x.shape[0]),) + ((0, 0),) * (x.ndim - 1))


def kernel(H, Z, block_id, batch_id, edges, params):
    H2 = _segsum(H, block_id, NB)
    cnt = _segsum(jnp.ones((N_ATOMS, 1), jnp.float32), block_id, NB)
    cnt = jnp.maximum(cnt, 1.0)
    H2 = H2 / cnt
    Zb = _segsum(Z, block_id, NB) / cnt

    src, dst = edges[0], edges[1]
    dvec = Zb[dst] - Zb[src]
    dist = jnp.sqrt(jnp.sum(dvec * dvec, axis=-1) + 1e-12)
    keep = dist > 0.01
    rbf = _rbf(dist)

    dst_key = jnp.where(keep, dst, NB)
    order = jnp.argsort(dst_key, stable=True)
    counts = _segsum(keep.astype(jnp.int32), dst, NB)
    offsets = jnp.concatenate([jnp.zeros((1,), counts.dtype),
                               jnp.cumsum(counts)[:-1]])
    max_rank = jnp.max(jnp.where(keep, counts[src], 0))
    counts_src = counts[src]
    base = offsets[src]

    u1 = dvec / dist[:, None]
    u2_tab = -u1
    rbf_p = _pad_rows(rbf, NE_PAD)
    srcf_tab = src.astype(jnp.float32)[:, None]
    u2_tab4 = jnp.pad(u2_tab, ((0, 0), (0, 1)))
    keepf = _pad_rows(keep.astype(jnp.float32)[:, None], NE_PAD)
    u1p = _pad_rows(jnp.pad(u1, ((0, 0), (0, 1))), NE_PAD)
    dstf = _pad_rows(dst.astype(jnp.float32)[:, None], NE_PAD)
    csf = _pad_rows(counts_src.astype(jnp.float32)[:, None], NE_PAD)
    base_p = _pad_rows(base[:, None], NE_PAD)[:, 0]

    Hs = _pad_rows(H2[src], NE_PAD)
    Hd = _pad_rows(H2[dst], NE_PAD)
    x = _emb_kernel(Hs, Hd, rbf_p, params['emb']['w'], params['emb']['b'])

    def out_block(p, x):
        y = _oute_kernel(x, rbf_p, p['rbf']['w'], keepf)[:NE]
        t = _segsum(y, dst, NB)
        t = _outb_kernel(_pad_rows(t, NB_PAD), p)[:NB]
        return t

    P = out_block(params['outs'][0], x)

    for b in range(NL):
        p = params['inter'][b]
        xji, xd = _pre_kernel(x, rbf_p, p)
        w12 = jnp.dot(p['sbf1']['w'], p['sbf2']['w'])
        w12r = jnp.transpose(w12.reshape(NSPH, NRAD, INT_EMB),
                             (1, 0, 2)).reshape(NRAD, NSPH * INT_EMB)
        tbl = jnp.concatenate(
            [xd[:NE], u2_tab, rbf, srcf_tab,
             jnp.zeros((NE, TBLW - INT_EMB - 4 - NRAD), jnp.float32)],
            axis=1)

        def body(state):
            r0, acc = state
            idx = jnp.clip(base_p[None, :] + r0
                           + jnp.arange(RC, dtype=r0.dtype)[:, None],
                           0, NE - 1)
            kj = jnp.concatenate(
                [order[idx].reshape(-1),
                 jnp.zeros((B_GATH - RC * NE_PAD,), jnp.int32)])
            gthr = _sc_gather(tbl, kj)
            r0f = r0.astype(jnp.float32).reshape(1, 1)
            acc = acc + _trip_chunk(gthr, u1p, dstf, keepf, csf, w12r, r0f)
            return r0 + RC, acc

        r0 = jnp.zeros((), max_rank.dtype)
        acc0 = jnp.zeros((NE_PAD, INT_EMB), jnp.float32)
        _, acc = jax.lax.while_loop(lambda s: s[0] < max_rank, body, (r0, acc0))
        x = _post_kernel(xji, acc, x, p)
        P = P + out_block(params['outs'][b + 1], x)

    block_repr = _l2n_kernel(_pad_rows(P, NB_PAD))[:NB]
    graph_repr = _l2n_kernel(_segsum(block_repr, batch_id, NG))
    return (H2, block_repr, graph_repr)


# speedup vs baseline: 5.3546x; 2.3367x over previous
import functools
import jax
import jax.numpy as jnp
import numpy as np
from jax.experimental import pallas as pl
from jax.experimental.pallas import tpu as pltpu

HID = 128; NL = 3; INT_EMB = 64; BASIS_EMB = 8; OUT_EMB = 256
NSPH = 7; NRAD = 6; CUT = 5.0
N_ATOMS = 40000; NB = 10000; NG = 64; NE = 80000

ET = 512
BT = 512
NE_PAD = ((NE + ET - 1) // ET) * ET
NB_PAD = ((NB + BT - 1) // BT) * BT


def _silu(x):
    return x * jax.nn.sigmoid(x)


def _cdiv(a, b):
    return (a + b - 1) // b


def _emb_body(hs_ref, hd_ref, rbf_ref, w1_ref, w2_ref, w3_ref, b_ref, o_ref):
    acc = jnp.dot(hs_ref[...], w1_ref[...], preferred_element_type=jnp.float32)
    acc += jnp.dot(hd_ref[...], w2_ref[...], preferred_element_type=jnp.float32)
    acc += jnp.dot(rbf_ref[...], w3_ref[...], preferred_element_type=jnp.float32)
    o_ref[...] = _silu(acc + b_ref[...])


def _emb_kernel(hs, hd, rbf, w, b):
    w1, w2, w3 = w[:HID], w[HID:2 * HID], w[2 * HID:]
    n = hs.shape[0]
    grid = (_cdiv(n, ET),)
    return pl.pallas_call(
        _emb_body,
        grid=grid,
        in_specs=[
            pl.BlockSpec((ET, HID), lambda i: (i, 0)),
            pl.BlockSpec((ET, HID), lambda i: (i, 0)),
            pl.BlockSpec((ET, NRAD), lambda i: (i, 0)),
            pl.BlockSpec((HID, HID), lambda i: (0, 0)),
            pl.BlockSpec((HID, HID), lambda i: (0, 0)),
            pl.BlockSpec((NRAD, HID), lambda i: (0, 0)),
            pl.BlockSpec((1, HID), lambda i: (0, 0)),
        ],
        out_specs=pl.BlockSpec((ET, HID), lambda i: (i, 0)),
        out_shape=jax.ShapeDtypeStruct((n, HID), jnp.float32),
    )(hs, hd, rbf, w1, w2, w3, b.reshape(1, HID))


def _pre_body(x_ref, rbf_ref, wji_ref, bji_ref, wkj_ref, bkj_ref,
              r1_ref, r2_ref, wdn_ref, xji_ref, xd_ref):
    x = x_ref[...]
    rbf_h = jnp.dot(jnp.dot(rbf_ref[...], r1_ref[...],
                            preferred_element_type=jnp.float32), r2_ref[...],
                    preferred_element_type=jnp.float32)
    xji = _silu(jnp.dot(x, wji_ref[...], preferred_element_type=jnp.float32)
                + bji_ref[...])
    xkj = _silu(jnp.dot(x, wkj_ref[...], preferred_element_type=jnp.float32)
                + bkj_ref[...])
    xkj = xkj * rbf_h
    xd_ref[...] = _silu(jnp.dot(xkj, wdn_ref[...],
                                preferred_element_type=jnp.float32))
    xji_ref[...] = xji


def _pre_kernel(x, rbf, p):
    n = x.shape[0]
    grid = (_cdiv(n, ET),)
    return pl.pallas_call(
        _pre_body,
        grid=grid,
        in_specs=[
            pl.BlockSpec((ET, HID), lambda i: (i, 0)),
            pl.BlockSpec((ET, NRAD), lambda i: (i, 0)),
            pl.BlockSpec((HID, HID), lambda i: (0, 0)),
            pl.BlockSpec((1, HID), lambda i: (0, 0)),
            pl.BlockSpec((HID, HID), lambda i: (0, 0)),
            pl.BlockSpec((1, HID), lambda i: (0, 0)),
            pl.BlockSpec((NRAD, BASIS_EMB), lambda i: (0, 0)),
            pl.BlockSpec((BASIS_EMB, HID), lambda i: (0, 0)),
            pl.BlockSpec((HID, INT_EMB), lambda i: (0, 0)),
        ],
        out_specs=[
            pl.BlockSpec((ET, HID), lambda i: (i, 0)),
            pl.BlockSpec((ET, INT_EMB), lambda i: (i, 0)),
        ],
        out_shape=[
            jax.ShapeDtypeStruct((n, HID), jnp.float32),
            jax.ShapeDtypeStruct((n, INT_EMB), jnp.float32),
        ],
    )(x, rbf, p['ji']['w'], p['ji']['b'].reshape(1, HID),
      p['kj']['w'], p['kj']['b'].reshape(1, HID),
      p['rbf1']['w'], p['rbf2']['w'], p['down']['w'])


def _post_body(xji_ref, agg_ref, x_ref, wup_ref,
               b0w_ref, b0b_ref, b1w_ref, b1b_ref,
               skw_ref, skb_ref,
               a0w_ref, a0b_ref, a1w_ref, a1b_ref,
               a2w_ref, a2b_ref, a3w_ref, a3b_ref, o_ref):
    h = xji_ref[...] + _silu(jnp.dot(agg_ref[...], wup_ref[...],
                                     preferred_element_type=jnp.float32))
    t = _silu(jnp.dot(h, b0w_ref[...], preferred_element_type=jnp.float32) + b0b_ref[...])
    h = h + _silu(jnp.dot(t, b1w_ref[...], preferred_element_type=jnp.float32) + b1b_ref[...])
    h = _silu(jnp.dot(h, skw_ref[...], preferred_element_type=jnp.float32) + skb_ref[...]) + x_ref[...]
    t = _silu(jnp.dot(h, a0w_ref[...], preferred_element_type=jnp.float32) + a0b_ref[...])
    h = h + _silu(jnp.dot(t, a1w_ref[...], preferred_element_type=jnp.float32) + a1b_ref[...])
    t = _silu(jnp.dot(h, a2w_ref[...], preferred_element_type=jnp.float32) + a2b_ref[...])
    h = h + _silu(jnp.dot(t, a3w_ref[...], preferred_element_type=jnp.float32) + a3b_ref[...])
    o_ref[...] = h


def _post_kernel(xji, agg, x, p):
    n = x.shape[0]
    grid = (_cdiv(n, ET),)
    hh = lambda: pl.BlockSpec((HID, HID), lambda i: (0, 0))
    bb = lambda: pl.BlockSpec((1, HID), lambda i: (0, 0))
    ws = [p['up']['w'],
          p['before'][0][0]['w'], p['before'][0][0]['b'].reshape(1, HID),
          p['before'][0][1]['w'], p['before'][0][1]['b'].reshape(1, HID),
          p['skip']['w'], p['skip']['b'].reshape(1, HID),
          p['after'][0][0]['w'], p['after'][0][0]['b'].reshape(1, HID),
          p['after'][0][1]['w'], p['after'][0][1]['b'].reshape(1, HID),
          p['after'][1][0]['w'], p['after'][1][0]['b'].reshape(1, HID),
          p['after'][1][1]['w'], p['after'][1][1]['b'].reshape(1, HID)]
    wspecs = [pl.BlockSpec((INT_EMB, HID), lambda i: (0, 0))]
    for k in range(1, len(ws)):
        wspecs.append(hh() if ws[k].shape[0] == HID else bb())
    return pl.pallas_call(
        _post_body,
        grid=grid,
        in_specs=[
            pl.BlockSpec((ET, HID), lambda i: (i, 0)),
            pl.BlockSpec((ET, INT_EMB), lambda i: (i, 0)),
            pl.BlockSpec((ET, HID), lambda i: (i, 0)),
        ] + wspecs,
        out_specs=pl.BlockSpec((ET, HID), lambda i: (i, 0)),
        out_shape=jax.ShapeDtypeStruct((n, HID), jnp.float32),
    )(xji, agg, x, *ws)


def _oute_body(x_ref, rbf_ref, wr_ref, keep_ref, o_ref):
    g = jnp.dot(rbf_ref[...], wr_ref[...], preferred_element_type=jnp.float32)
    o_ref[...] = jnp.where(keep_ref[...] > 0, g * x_ref[...], 0.0)


def _oute_kernel(x, rbf, wr, keepf):
    n = x.shape[0]
    grid = (_cdiv(n, ET),)
    return pl.pallas_call(
        _oute_body,
        grid=grid,
        in_specs=[
            pl.BlockSpec((ET, HID), lambda i: (i, 0)),
            pl.BlockSpec((ET, NRAD), lambda i: (i, 0)),
            pl.BlockSpec((NRAD, HID), lambda i: (0, 0)),
            pl.BlockSpec((ET, 1), lambda i: (i, 0)),
        ],
        out_specs=pl.BlockSpec((ET, HID), lambda i: (i, 0)),
        out_shape=jax.ShapeDtypeStruct((n, HID), jnp.float32),
    )(x, rbf, wr, keepf)


def _outb_body(t_ref, wu_ref, l0w_ref, l0b_ref, l1w_ref, l1b_ref,
               l2w_ref, l2b_ref, wo_ref, o_ref):
    t = jnp.dot(t_ref[...], wu_ref[...], preferred_element_type=jnp.float32)
    t = _silu(jnp.dot(t, l0w_ref[...], preferred_element_type=jnp.float32) + l0b_ref[...])
    t = _silu(jnp.dot(t, l1w_ref[...], preferred_element_type=jnp.float32) + l1b_ref[...])
    t = _silu(jnp.dot(t, l2w_ref[...], preferred_element_type=jnp.float32) + l2b_ref[...])
    o_ref[...] = jnp.dot(t, wo_ref[...], preferred_element_type=jnp.float32)


def _outb_kernel(t, p):
    n = t.shape[0]
    grid = (_cdiv(n, BT),)
    oo = lambda: pl.BlockSpec((OUT_EMB, OUT_EMB), lambda i: (0, 0))
    ob = lambda: pl.BlockSpec((1, OUT_EMB), lambda i: (0, 0))
    return pl.pallas_call(
        _outb_body,
        grid=grid,
        in_specs=[
            pl.BlockSpec((BT, HID), lambda i: (i, 0)),
            pl.BlockSpec((HID, OUT_EMB), lambda i: (0, 0)),
            oo(), ob(), oo(), ob(), oo(), ob(),
            pl.BlockSpec((OUT_EMB, HID), lambda i: (0, 0)),
        ],
        out_specs=pl.BlockSpec((BT, HID), lambda i: (i, 0)),
        out_shape=jax.ShapeDtypeStruct((n, HID), jnp.float32),
    )(t, p['up']['w'],
      p['lins'][0]['w'], p['lins'][0]['b'].reshape(1, OUT_EMB),
      p['lins'][1]['w'], p['lins'][1]['b'].reshape(1, OUT_EMB),
      p['lins'][2]['w'], p['lins'][2]['b'].reshape(1, OUT_EMB),
      p['out']['w'])


NTILES = NE_PAD // ET


def _trip_body(xdg_ref, u2g_ref, rbg_ref, srcg_ref, u1_ref, dstf_ref,
               keepf_ref, csf_ref, w12r_ref, rf_ref, accin_ref, acc_ref):
    u2g = u2g_ref[...]
    rbg = rbg_ref[...]
    r_f = rf_ref[0, 0]
    c = jnp.sum(u1_ref[...][:, 0:3] * u2g[:, 0:3], axis=-1, keepdims=True)
    rbw = jnp.dot(rbg, w12r_ref[...], preferred_element_type=jnp.float32)
    tl0 = jnp.ones_like(c)
    tl1 = c
    sbf_h = rbw[:, 0:INT_EMB] + c * rbw[:, INT_EMB:2 * INT_EMB]
    for l in range(2, NSPH):
        tl0, tl1 = tl1, 2.0 * c * tl1 - tl0
        sbf_h += tl1 * rbw[:, l * INT_EMB:(l + 1) * INT_EMB]
    valid = ((keepf_ref[...] > 0) & (r_f < csf_ref[...])
             & (srcg_ref[...] != dstf_ref[...]))
    acc_ref[...] = accin_ref[...] + jnp.where(valid, xdg_ref[...] * sbf_h, 0.0)


def _trip_rank(xdg, u2g, rbg, srcg, u1, dstf, keepf, csf, w12r, rf, acc):
    blk = lambda w: pl.BlockSpec((ET, w), lambda i: (i, 0))
    return pl.pallas_call(
        _trip_body,
        grid=(NTILES,),
        in_specs=[
            blk(INT_EMB), blk(4), blk(NRAD), blk(1),
            blk(4), blk(1), blk(1), blk(1),
            pl.BlockSpec((NRAD, NSPH * INT_EMB), lambda i: (0, 0)),
            pl.BlockSpec((1, 1), lambda i: (0, 0)),
            blk(INT_EMB),
        ],
        out_specs=pl.BlockSpec((ET, INT_EMB), lambda i: (i, 0)),
        out_shape=jax.ShapeDtypeStruct((NE_PAD, INT_EMB), jnp.float32),
        input_output_aliases={10: 0},
    )(xdg, u2g, rbg, srcg, u1, dstf, keepf, csf, w12r, rf, acc)


def _l2n_body(x_ref, o_ref):
    x = x_ref[...]
    nrm = jnp.sqrt(jnp.sum(x * x, axis=-1, keepdims=True))
    o_ref[...] = x / jnp.maximum(nrm, 1e-12)


def _l2n_kernel(x):
    n, d = x.shape
    bt = min(n, BT)
    grid = (_cdiv(n, bt),)
    return pl.pallas_call(
        _l2n_body,
        grid=grid,
        in_specs=[pl.BlockSpec((bt, d), lambda i: (i, 0))],
        out_specs=pl.BlockSpec((bt, d), lambda i: (i, 0)),
        out_shape=jax.ShapeDtypeStruct((n, d), jnp.float32),
    )(x)


def _envelope(x):
    p = 6
    a = -(p + 1) * (p + 2) / 2.0
    b = p * (p + 2)
    cc = -p * (p + 1) / 2.0
    xs = jnp.maximum(x, 1e-9)
    env = 1.0 / xs + a * xs ** (p - 1) + b * xs ** p + cc * xs ** (p + 1)
    return jnp.where(x < 1.0, env, 0.0)


def _rbf(d):
    x = d / CUT
    freqs = jnp.arange(1, NRAD + 1, dtype=jnp.float32) * jnp.pi
    return _envelope(x)[:, None] * jnp.sin(freqs[None, :] * x[:, None])


def _segsum(x, idx, n):
    return jax.ops.segment_sum(x, idx, num_segments=n)


def _pad_rows(x, n):
    return jnp.pad(x, ((0, n - x.shape[0]),) + ((0, 0),) * (x.ndim - 1))


def kernel(H, Z, block_id, batch_id, edges, params):
    H2 = _segsum(H, block_id, NB)
    cnt = _segsum(jnp.ones((N_ATOMS, 1), jnp.float32), block_id, NB)
    cnt = jnp.maximum(cnt, 1.0)
    H2 = H2 / cnt
    Zb = _segsum(Z, block_id, NB) / cnt

    src, dst = edges[0], edges[1]
    dvec = Zb[dst] - Zb[src]
    dist = jnp.sqrt(jnp.sum(dvec * dvec, axis=-1) + 1e-12)
    keep = dist > 0.01
    rbf = _rbf(dist)

    dst_key = jnp.where(keep, dst, NB)
    order = jnp.argsort(dst_key, stable=True)
    counts = _segsum(keep.astype(jnp.int32), dst, NB)
    offsets = jnp.concatenate([jnp.zeros((1,), counts.dtype),
                               jnp.cumsum(counts)[:-1]])
    max_rank = jnp.max(jnp.where(keep, counts[src], 0))
    counts_src = counts[src]
    base = offsets[src]

    u1 = dvec / dist[:, None]
    u2_tab = -u1
    rbf_p = _pad_rows(rbf, NE_PAD)
    srcf_tab = src.astype(jnp.float32)[:, None]
    u2_tab4 = jnp.pad(u2_tab, ((0, 0), (0, 1)))
    keepf = _pad_rows(keep.astype(jnp.float32)[:, None], NE_PAD)
    u1p = _pad_rows(jnp.pad(u1, ((0, 0), (0, 1))), NE_PAD)
    dstf = _pad_rows(dst.astype(jnp.float32)[:, None], NE_PAD)
    csf = _pad_rows(counts_src.astype(jnp.float32)[:, None], NE_PAD)
    base_p = _pad_rows(base[:, None], NE_PAD)[:, 0]

    Hs = _pad_rows(H2[src], NE_PAD)
    Hd = _pad_rows(H2[dst], NE_PAD)
    x = _emb_kernel(Hs, Hd, rbf_p, params['emb']['w'], params['emb']['b'])

    def out_block(p, x):
        y = _oute_kernel(x, rbf_p, p['rbf']['w'], keepf)[:NE]
        t = _segsum(y, dst, NB)
        t = _outb_kernel(_pad_rows(t, NB_PAD), p)[:NB]
        return t

    P = out_block(params['outs'][0], x)

    for b in range(NL):
        p = params['inter'][b]
        xji, xd = _pre_kernel(x, rbf_p, p)
        w12 = jnp.dot(p['sbf1']['w'], p['sbf2']['w'])
        w12r = jnp.transpose(w12.reshape(NSPH, NRAD, INT_EMB),
                             (1, 0, 2)).reshape(NRAD, NSPH * INT_EMB)
        xd_ne = xd[:NE]

        def body(state):
            r, acc = state
            kj = order[jnp.clip(base_p + r, 0, NE - 1)]
            rf = r.astype(jnp.float32).reshape(1, 1)
            acc = _trip_rank(xd_ne[kj], u2_tab4[kj], rbf[kj], srcf_tab[kj],
                             u1p, dstf, keepf, csf, w12r, rf, acc)
            return r + 1, acc

        r0 = jnp.zeros((), max_rank.dtype)
        acc0 = jnp.zeros((NE_PAD, INT_EMB), jnp.float32)
        _, acc = jax.lax.while_loop(lambda s: s[0] < max_rank, body, (r0, acc0))
        x = _post_kernel(xji, acc, x, p)
        P = P + out_block(params['outs'][b + 1], x)

    block_repr = _l2n_kernel(_pad_rows(P, NB_PAD))[:NB]
    graph_repr = _l2n_kernel(_segsum(block_repr, batch_id, NG))
    return (H2, block_repr, graph_repr)
